# baseline (device time: 44184 ns/iter reference)
import jax
import jax.numpy as jnp
from jax import lax
from jax.experimental import pallas as pl
from jax.experimental.pallas import tpu as pltpu

N_DEV = 4
NB = 4
N_SLOTS = 2 * (N_DEV - 1)


def kernel(x, w_mat):
    m, k_per = x.shape
    _, n = w_mat.shape
    mc = m // N_DEV
    mcb = mc // NB
    half = n // 2

    def body(x_ref, w_ref, out_ref,
             comm_r, comm_l, send_r, recv_r, send_l, recv_l):
        my = lax.axis_index("i")
        left = lax.rem(my + N_DEV - 1, N_DEV)
        right = lax.rem(my + 1, N_DEV)
        cols = {"r": pl.ds(0, half), "l": pl.ds(half, half)}
        comm = {"r": comm_r, "l": comm_l}
        ssem = {"r": send_r, "l": send_l}
        rsem = {"r": recv_r, "l": recv_l}
        peer = {"r": right, "l": left}
        sgn = {"r": -1, "l": +1}
        descs = {}

        def chunk_rows(c, b):
            return pl.ds(c * mc + b * mcb, mcb)

        def start(d, slot, b, src_ref):
            r = pltpu.make_async_remote_copy(
                src_ref=src_ref,
                dst_ref=comm[d].at[slot, pl.ds(b * mcb, mcb), :],
                send_sem=ssem[d].at[slot * NB + b],
                recv_sem=rsem[d].at[slot * NB + b],
                device_id=(peer[d],),
                device_id_type=pl.DeviceIdType.MESH,
            )
            r.start()
            descs[d, slot, b] = r

        barrier_sem = pltpu.get_barrier_semaphore()
        for nbr in (left, right):
            pl.semaphore_signal(
                barrier_sem, inc=1,
                device_id=(nbr,), device_id_type=pl.DeviceIdType.MESH,
            )
        pl.semaphore_wait(barrier_sem, 2)

        def gemm_chunk(c):
            sl = pl.ds(c * mc, mc)
            out_ref[sl, :] = jnp.dot(
                x_ref[sl, :], w_ref[...], preferred_element_type=jnp.float32
            )

        gemm_chunk(my)
        for b in range(NB):
            for d in ("r", "l"):
                start(d, 0, b, out_ref.at[chunk_rows(my, b), cols[d]])
        for k in range(1, N_DEV):
            gemm_chunk(lax.rem(my + k, N_DEV))

        for s in range(1, N_DEV - 1):
            for b in range(NB):
                for d in ("r", "l"):
                    c = lax.rem(my + sgn[d] * s + N_DEV, N_DEV)
                    rows = chunk_rows(c, b)
                    descs[d, s - 1, b].wait_recv()
                    out_ref[rows, cols[d]] = (
                        out_ref[rows, cols[d]]
                        + comm[d][s - 1, pl.ds(b * mcb, mcb), :]
                    )
                    start(d, s, b, out_ref.at[rows, cols[d]])

        for b in range(NB):
            for d in ("r", "l"):
                own = lax.rem(my - sgn[d] + N_DEV, N_DEV)
                rows = chunk_rows(own, b)
                descs[d, N_DEV - 2, b].wait_recv()
                y = out_ref[rows, cols[d]] + comm[d][
                    N_DEV - 2, pl.ds(b * mcb, mcb), :
                ]
                out_ref[rows, cols[d]] = y * jax.nn.sigmoid(y)
                start(d, N_DEV - 1, b, out_ref.at[rows, cols[d]])

        for t in range(N_DEV - 1):
            ag = N_DEV - 1 + t
            for b in range(NB):
                for d in ("r", "l"):
                    c = lax.rem(my + sgn[d] * t + N_DEV, N_DEV)
                    descs[d, ag, b].wait_recv()
                    if t < N_DEV - 2:
                        start(d, ag + 1, b,
                              comm[d].at[ag, pl.ds(b * mcb, mcb), :])
                    descs[d, t, b].wait_send()
                    out_ref[chunk_rows(c, b), cols[d]] = comm[d][
                        ag, pl.ds(b * mcb, mcb), :
                    ]

        for t in range(N_DEV - 1):
            for b in range(NB):
                for d in ("r", "l"):
                    descs[d, N_DEV - 1 + t, b].wait_send()

    return pl.pallas_call(
        body,
        out_shape=jax.ShapeDtypeStruct((m, n), jnp.float32),
        in_specs=[
            pl.BlockSpec(memory_space=pltpu.VMEM),
            pl.BlockSpec(memory_space=pltpu.VMEM),
        ],
        out_specs=pl.BlockSpec(memory_space=pltpu.VMEM),
        scratch_shapes=[
            pltpu.VMEM((N_SLOTS, mc, half), jnp.float32),
            pltpu.VMEM((N_SLOTS, mc, half), jnp.float32),
            pltpu.SemaphoreType.DMA((N_SLOTS * NB,)),
            pltpu.SemaphoreType.DMA((N_SLOTS * NB,)),
            pltpu.SemaphoreType.DMA((N_SLOTS * NB,)),
            pltpu.SemaphoreType.DMA((N_SLOTS * NB,)),
        ],
        compiler_params=pltpu.CompilerParams(collective_id=0),
    )(x, w_mat)


# device time: 28177 ns/iter; 1.5681x vs baseline; 1.5681x over previous
import jax
import jax.numpy as jnp
from jax import lax
from jax.experimental import pallas as pl
from jax.experimental.pallas import tpu as pltpu

N_DEV = 4
NB = 2
N_SLOTS = 2 * (N_DEV - 1)


def kernel(x, w_mat):
    m, k_per = x.shape
    _, n = w_mat.shape
    mc = m // N_DEV
    mcb = mc // NB
    half = n // 2

    def body(x_ref, w_ref, out_ref,
             comm_r, comm_l, stage_r, stage_l,
             send_r, recv_r, send_l, recv_l):
        my = lax.axis_index("i")
        left = lax.rem(my + N_DEV - 1, N_DEV)
        right = lax.rem(my + 1, N_DEV)
        cols = {"r": pl.ds(0, half), "l": pl.ds(half, half)}
        comm = {"r": comm_r, "l": comm_l}
        stage = {"r": stage_r, "l": stage_l}
        ssem = {"r": send_r, "l": send_l}
        rsem = {"r": recv_r, "l": recv_l}
        peer = {"r": right, "l": left}
        sgn = {"r": -1, "l": +1}
        descs = {}

        def chunk_rows(c, b):
            return pl.ds(c * mc + b * mcb, mcb)

        def blk(slot, b):
            return (slot, pl.ds(b * mcb, mcb), slice(None))

        def start(d, slot, b, src_ref):
            r = pltpu.make_async_remote_copy(
                src_ref=src_ref,
                dst_ref=comm[d].at[blk(slot, b)],
                send_sem=ssem[d].at[slot * NB + b],
                recv_sem=rsem[d].at[slot * NB + b],
                device_id=(peer[d],),
                device_id_type=pl.DeviceIdType.MESH,
            )
            r.start()
            descs[d, slot, b] = r

        barrier_sem = pltpu.get_barrier_semaphore()
        for nbr in (left, right):
            pl.semaphore_signal(
                barrier_sem, inc=1,
                device_id=(nbr,), device_id_type=pl.DeviceIdType.MESH,
            )
        pl.semaphore_wait(barrier_sem, 2)

        def gemm_chunk(c):
            sl = pl.ds(c * mc, mc)
            out_ref[sl, :] = jnp.dot(
                x_ref[sl, :], w_ref[...], preferred_element_type=jnp.float32
            )

        gemm_chunk(my)
        for b in range(NB):
            for d in ("r", "l"):
                stage[d][blk(0, b)] = out_ref[
                    chunk_rows(my, b), cols[d]
                ].astype(jnp.bfloat16)
                start(d, 0, b, stage[d].at[blk(0, b)])
        for k in range(1, N_DEV):
            gemm_chunk(lax.rem(my + k, N_DEV))

        for s in range(1, N_DEV - 1):
            for b in range(NB):
                for d in ("r", "l"):
                    c = lax.rem(my + sgn[d] * s + N_DEV, N_DEV)
                    rows = chunk_rows(c, b)
                    descs[d, s - 1, b].wait_recv()
                    acc = out_ref[rows, cols[d]] + comm[d][
                        blk(s - 1, b)
                    ].astype(jnp.float32)
                    out_ref[rows, cols[d]] = acc
                    stage[d][blk(s, b)] = acc.astype(jnp.bfloat16)
                    start(d, s, b, stage[d].at[blk(s, b)])

        for b in range(NB):
            for d in ("r", "l"):
                own = lax.rem(my - sgn[d] + N_DEV, N_DEV)
                rows = chunk_rows(own, b)
                descs[d, N_DEV - 2, b].wait_recv()
                y = out_ref[rows, cols[d]] + comm[d][
                    blk(N_DEV - 2, b)
                ].astype(jnp.float32)
                y = y * jax.nn.sigmoid(y)
                out_ref[rows, cols[d]] = y
                stage[d][blk(N_DEV - 1, b)] = y.astype(jnp.bfloat16)
                start(d, N_DEV - 1, b, stage[d].at[blk(N_DEV - 1, b)])

        for t in range(N_DEV - 1):
            ag = N_DEV - 1 + t
            for b in range(NB):
                for d in ("r", "l"):
                    c = lax.rem(my + sgn[d] * t + N_DEV, N_DEV)
                    descs[d, ag, b].wait_recv()
                    if t < N_DEV - 2:
                        start(d, ag + 1, b, comm[d].at[blk(ag, b)])
                    out_ref[chunk_rows(c, b), cols[d]] = comm[d][
                        blk(ag, b)
                    ].astype(jnp.float32)

        for slot in range(N_SLOTS):
            for b in range(NB):
                for d in ("r", "l"):
                    descs[d, slot, b].wait_send()

    return pl.pallas_call(
        body,
        out_shape=jax.ShapeDtypeStruct((m, n), jnp.float32),
        in_specs=[
            pl.BlockSpec(memory_space=pltpu.VMEM),
            pl.BlockSpec(memory_space=pltpu.VMEM),
        ],
        out_specs=pl.BlockSpec(memory_space=pltpu.VMEM),
        scratch_shapes=[
            pltpu.VMEM((N_SLOTS, mc, half), jnp.bfloat16),
            pltpu.VMEM((N_SLOTS, mc, half), jnp.bfloat16),
            pltpu.VMEM((N_SLOTS, mc, half), jnp.bfloat16),
            pltpu.VMEM((N_SLOTS, mc, half), jnp.bfloat16),
            pltpu.SemaphoreType.DMA((N_SLOTS * NB,)),
            pltpu.SemaphoreType.DMA((N_SLOTS * NB,)),
            pltpu.SemaphoreType.DMA((N_SLOTS * NB,)),
            pltpu.SemaphoreType.DMA((N_SLOTS * NB,)),
        ],
        compiler_params=pltpu.CompilerParams(collective_id=0),
    )(x, w_mat)


# device time: 27350 ns/iter; 1.6155x vs baseline; 1.0302x over previous
import jax
import jax.numpy as jnp
from jax import lax
from jax.experimental import pallas as pl
from jax.experimental.pallas import tpu as pltpu

N_DEV = 4
NB = 4
N_SLOTS = 2 * (N_DEV - 1)


def kernel(x, w_mat):
    m, k_per = x.shape
    _, n = w_mat.shape
    mc = m // N_DEV
    mcb = mc // NB
    half = n // 2

    def body(x_ref, w_ref, out_ref,
             comm_r, comm_l, stage_r, stage_l,
             send_r, recv_r, send_l, recv_l):
        my = lax.axis_index("i")
        left = lax.rem(my + N_DEV - 1, N_DEV)
        right = lax.rem(my + 1, N_DEV)
        cols = {"r": pl.ds(0, half), "l": pl.ds(half, half)}
        comm = {"r": comm_r, "l": comm_l}
        stage = {"r": stage_r, "l": stage_l}
        ssem = {"r": send_r, "l": send_l}
        rsem = {"r": recv_r, "l": recv_l}
        peer = {"r": right, "l": left}
        sgn = {"r": -1, "l": +1}
        descs = {}

        def chunk_rows(c, b):
            return pl.ds(c * mc + b * mcb, mcb)

        def blk(slot, b):
            return (slot, pl.ds(b * mcb, mcb), slice(None))

        def start(d, slot, b, src_ref):
            r = pltpu.make_async_remote_copy(
                src_ref=src_ref,
                dst_ref=comm[d].at[blk(slot, b)],
                send_sem=ssem[d].at[slot * NB + b],
                recv_sem=rsem[d].at[slot * NB + b],
                device_id=(peer[d],),
                device_id_type=pl.DeviceIdType.MESH,
            )
            r.start()
            descs[d, slot, b] = r

        barrier_sem = pltpu.get_barrier_semaphore()
        for nbr in (left, right):
            pl.semaphore_signal(
                barrier_sem, inc=1,
                device_id=(nbr,), device_id_type=pl.DeviceIdType.MESH,
            )
        pl.semaphore_wait(barrier_sem, 2)

        def gemm_chunk(c):
            sl = pl.ds(c * mc, mc)
            out_ref[sl, :] = jnp.dot(
                x_ref[sl, :], w_ref[...], preferred_element_type=jnp.float32
            )

        gemm_chunk(my)
        for b in range(NB):
            for d in ("r", "l"):
                stage[d][blk(0, b)] = out_ref[
                    chunk_rows(my, b), cols[d]
                ].astype(jnp.bfloat16)
                start(d, 0, b, stage[d].at[blk(0, b)])
        for k in range(1, N_DEV):
            gemm_chunk(lax.rem(my + k, N_DEV))

        for s in range(1, N_DEV - 1):
            for b in range(NB):
                for d in ("r", "l"):
                    c = lax.rem(my + sgn[d] * s + N_DEV, N_DEV)
                    rows = chunk_rows(c, b)
                    descs[d, s - 1, b].wait_recv()
                    acc = out_ref[rows, cols[d]] + comm[d][
                        blk(s - 1, b)
                    ].astype(jnp.float32)
                    out_ref[rows, cols[d]] = acc
                    stage[d][blk(s, b)] = acc.astype(jnp.bfloat16)
                    start(d, s, b, stage[d].at[blk(s, b)])

        for b in range(NB):
            for d in ("r", "l"):
                own = lax.rem(my - sgn[d] + N_DEV, N_DEV)
                rows = chunk_rows(own, b)
                descs[d, N_DEV - 2, b].wait_recv()
                y = out_ref[rows, cols[d]] + comm[d][
                    blk(N_DEV - 2, b)
                ].astype(jnp.float32)
                y = y * jax.nn.sigmoid(y)
                out_ref[rows, cols[d]] = y
                stage[d][blk(N_DEV - 1, b)] = y.astype(jnp.bfloat16)
                start(d, N_DEV - 1, b, stage[d].at[blk(N_DEV - 1, b)])

        for t in range(N_DEV - 1):
            ag = N_DEV - 1 + t
            for b in range(NB):
                for d in ("r", "l"):
                    c = lax.rem(my + sgn[d] * t + N_DEV, N_DEV)
                    descs[d, ag, b].wait_recv()
                    if t < N_DEV - 2:
                        start(d, ag + 1, b, comm[d].at[blk(ag, b)])
                    out_ref[chunk_rows(c, b), cols[d]] = comm[d][
                        blk(ag, b)
                    ].astype(jnp.float32)

        for slot in range(N_SLOTS):
            for b in range(NB):
                for d in ("r", "l"):
                    descs[d, slot, b].wait_send()

    return pl.pallas_call(
        body,
        out_shape=jax.ShapeDtypeStruct((m, n), jnp.float32),
        in_specs=[
            pl.BlockSpec(memory_space=pltpu.VMEM),
            pl.BlockSpec(memory_space=pltpu.VMEM),
        ],
        out_specs=pl.BlockSpec(memory_space=pltpu.VMEM),
        scratch_shapes=[
            pltpu.VMEM((N_SLOTS, mc, half), jnp.bfloat16),
            pltpu.VMEM((N_SLOTS, mc, half), jnp.bfloat16),
            pltpu.VMEM((N_SLOTS, mc, half), jnp.bfloat16),
            pltpu.VMEM((N_SLOTS, mc, half), jnp.bfloat16),
            pltpu.SemaphoreType.DMA((N_SLOTS * NB,)),
            pltpu.SemaphoreType.DMA((N_SLOTS * NB,)),
            pltpu.SemaphoreType.DMA((N_SLOTS * NB,)),
            pltpu.SemaphoreType.DMA((N_SLOTS * NB,)),
        ],
        compiler_params=pltpu.CompilerParams(collective_id=0),
    )(x, w_mat)
